# Initial kernel scaffold; baseline (speedup 1.0000x reference)
import jax
import jax.numpy as jnp
from jax.experimental import pallas as pl


def kernel(val, remain_padding_mask, revert_idx, mask_token, pos_enc):
    raise NotImplementedError("write your pallas kernel here")


# speedup vs baseline: 5.3946x; 5.3946x over previous
import functools

import jax
import jax.numpy as jnp
from jax import lax
from jax.experimental import pallas as pl
from jax.experimental.pallas import tpu as pltpu
from jax.experimental.pallas import tpu_sc as plsc

B, LK, LF, D = 16, 1024, 2048, 768
NC, NS = 2, 16
NW = NC * NS
ROWS = B * LF
LPW = LF // NW
C = 32
MT_ROWS = 8
VPR = D // 16


def _body(table, ridx, mask, pos, out, pos_res, idxr_a, idxm_a, mch_a, idxg_a,
          g0, g1, sem):
    wid = lax.axis_index("s") * NC + lax.axis_index("c")
    l0 = wid * LPW

    pltpu.sync_copy(pos.at[pl.ds(l0, LPW)], pos_res)
    descs = [pltpu.async_copy(ridx.at[pl.ds(b * LF + l0, LPW)],
                              idxr_a.at[b], sem) for b in range(B)]
    for d in descs:
        d.wait()

    for k in range(B * LPW // 16):
        b = k // (LPW // 16)
        i = idxr_a[b, pl.ds((k % (LPW // 16)) * 16, 16)]
        idxm_a[pl.ds(k * 16, 16)] = b * LK + jnp.minimum(i, LK - 1)
    for j in range(B * LPW // 128):
        pltpu.async_copy(mask.at[idxm_a.at[pl.ds(j * 128, 128)]],
                         mch_a.at[pl.ds(j * 128, 128)], sem).wait()
    for k in range(B * LPW // 16):
        b = k // (LPW // 16)
        i = idxr_a[b, pl.ds((k % (LPW // 16)) * 16, 16)]
        mv = mch_a[pl.ds(k * 16, 16)]
        valid = (i < LK) & (mv == 1)
        idxg_a[pl.ds(k * 16, 16)] = jnp.where(
            valid, b * LK + i, B * LK + (i & (MT_ROWS - 1)))

    def run_chunk(b, half, gbuf):
        base = b * LF + l0 + half * C
        ioff = b * LPW + half * C
        pltpu.async_copy(table.at[idxg_a.at[pl.ds(ioff, C)]], gbuf, sem).wait()

        def row(r, carry):
            for j in range(VPR):
                s = pl.ds(j * 16, 16)
                gbuf[r, s] = gbuf[r, s] + pos_res[half * C + r, s]
            return carry

        lax.fori_loop(0, C, row, 0)
        pltpu.sync_copy(gbuf, out.at[pl.ds(base, C)])

    def batch(b, carry):
        run_chunk(b, 0, g0)
        run_chunk(b, 1, g1)
        return carry

    lax.fori_loop(0, B, batch, 0)


@functools.partial(
    pl.kernel,
    out_type=jax.ShapeDtypeStruct((ROWS, D), jnp.float32),
    mesh=plsc.VectorSubcoreMesh(core_axis_name="c", subcore_axis_name="s",
                                num_cores=NC, num_subcores=NS),
    scratch_types=[
        pltpu.VMEM((LPW, D), jnp.float32),
        pltpu.VMEM((B, LPW), jnp.int32),
        pltpu.VMEM((B * LPW,), jnp.int32),
        pltpu.VMEM((B * LPW,), jnp.int32),
        pltpu.VMEM((B * LPW,), jnp.int32),
        pltpu.VMEM((C, D), jnp.float32),
        pltpu.VMEM((C, D), jnp.float32),
        pltpu.SemaphoreType.DMA,
    ],
)
def _revert_sc(table, ridx, mask, pos, out, pos_res, idxr_a, idxm_a, mch_a,
               idxg_a, g0, g1, sem):
    _body(table, ridx, mask, pos, out, pos_res, idxr_a, idxm_a, mch_a, idxg_a,
          g0, g1, sem)


def kernel(val, remain_padding_mask, revert_idx, mask_token, pos_enc):
    val2d = val.reshape(B * LK, D)
    mt = jnp.broadcast_to(mask_token[None, :].astype(jnp.float32), (MT_ROWS, D))
    table = jnp.concatenate([val2d, mt], axis=0)
    ridx = revert_idx.reshape(ROWS).astype(jnp.int32)
    mask = remain_padding_mask.reshape(B * LK).astype(jnp.int32)
    out = _revert_sc(table, ridx, mask, pos_enc.astype(jnp.float32))
    return out.reshape(B, LF, D)


# speedup vs baseline: 5.9608x; 1.1050x over previous
import functools

import jax
import jax.numpy as jnp
from jax import lax
from jax.experimental import pallas as pl
from jax.experimental.pallas import tpu as pltpu
from jax.experimental.pallas import tpu_sc as plsc

B, LK, LF, D = 16, 1024, 2048, 768
NC, NS = 2, 16
NW = NC * NS
ROWS = B * LF
LPW = LF // NW
C = 32
NCH = B * LPW // C
MT_ROWS = 8
VPR = D // 16


def _body(table, ridx, mask, pos, out, pos_res, idxr_a, idxm_a, mch_a, idxg_a,
          g0, g1, sem, sem_pos, sem_in0, sem_in1, sem_out0, sem_out1):
    wid = lax.axis_index("s") * NC + lax.axis_index("c")
    l0 = wid * LPW

    dpos = pltpu.async_copy(pos.at[pl.ds(l0, LPW)], pos_res, sem_pos)
    descs = [pltpu.async_copy(ridx.at[pl.ds(b * LF + l0, LPW)],
                              idxr_a.at[b], sem) for b in range(B)]
    for d in descs:
        d.wait()
    for k in range(B * LPW // 16):
        b = k // (LPW // 16)
        i = idxr_a[b, pl.ds((k % (LPW // 16)) * 16, 16)]
        idxm_a[pl.ds(k * 16, 16)] = b * LK + jnp.minimum(i, LK - 1)
    mdescs = [pltpu.async_copy(mask.at[idxm_a.at[pl.ds(j * 128, 128)]],
                               mch_a.at[pl.ds(j * 128, 128)], sem)
              for j in range(B * LPW // 128)]
    for d in mdescs:
        d.wait()
    for k in range(B * LPW // 16):
        b = k // (LPW // 16)
        i = idxr_a[b, pl.ds((k % (LPW // 16)) * 16, 16)]
        mv = mch_a[pl.ds(k * 16, 16)]
        valid = (i < LK) & (mv == 1)
        idxg_a[pl.ds(k * 16, 16)] = jnp.where(
            valid, b * LK + i, B * LK + (i & (MT_ROWS - 1)))
    dpos.wait()

    def out_base(c):
        return (c // 2) * LF + l0 + (c & 1) * C

    def gather(c, gbuf, sem_in):
        return pltpu.make_async_copy(
            table.at[idxg_a.at[pl.ds(c * C, C)]], gbuf, sem_in)

    def write(c, gbuf, sem_out):
        return pltpu.make_async_copy(gbuf, out.at[pl.ds(out_base(c), C)],
                                     sem_out)

    def valu(c, gbuf):
        poff = (c & 1) * C

        def row(r, carry):
            for j in range(VPR):
                s = pl.ds(j * 16, 16)
                gbuf[r, s] = gbuf[r, s] + pos_res[poff + r, s]
            return carry

        lax.fori_loop(0, C, row, 0)

    gather(0, g0, sem_in0).start()

    def step(c2, carry):
        c = c2 * 2
        @pl.when(c2 > 0)
        def _():
            write(c - 1, g1, sem_out1).wait()
        gather(c + 1, g1, sem_in1).start()
        gather(c, g0, sem_in0).wait()
        valu(c, g0)
        write(c, g0, sem_out0).start()
        write(c, g0, sem_out0).wait()

        @pl.when(c2 < NCH // 2 - 1)
        def _():
            gather(c + 2, g0, sem_in0).start()
        gather(c + 1, g1, sem_in1).wait()
        valu(c + 1, g1)
        write(c + 1, g1, sem_out1).start()
        return carry

    lax.fori_loop(0, NCH // 2, step, 0)
    write(NCH - 1, g1, sem_out1).wait()


@functools.partial(
    pl.kernel,
    out_type=jax.ShapeDtypeStruct((ROWS, D), jnp.float32),
    mesh=plsc.VectorSubcoreMesh(core_axis_name="c", subcore_axis_name="s",
                                num_cores=NC, num_subcores=NS),
    scratch_types=[
        pltpu.VMEM((LPW, D), jnp.float32),
        pltpu.VMEM((B, LPW), jnp.int32),
        pltpu.VMEM((B * LPW,), jnp.int32),
        pltpu.VMEM((B * LPW,), jnp.int32),
        pltpu.VMEM((B * LPW,), jnp.int32),
        pltpu.VMEM((C, D), jnp.float32),
        pltpu.VMEM((C, D), jnp.float32),
        pltpu.SemaphoreType.DMA,
        pltpu.SemaphoreType.DMA,
        pltpu.SemaphoreType.DMA,
        pltpu.SemaphoreType.DMA,
        pltpu.SemaphoreType.DMA,
        pltpu.SemaphoreType.DMA,
    ],
)
def _revert_sc(table, ridx, mask, pos, out, pos_res, idxr_a, idxm_a, mch_a,
               idxg_a, g0, g1, sem, sem_pos, sem_in0, sem_in1, sem_out0,
               sem_out1):
    _body(table, ridx, mask, pos, out, pos_res, idxr_a, idxm_a, mch_a, idxg_a,
          g0, g1, sem, sem_pos, sem_in0, sem_in1, sem_out0, sem_out1)


def kernel(val, remain_padding_mask, revert_idx, mask_token, pos_enc):
    val2d = val.reshape(B * LK, D)
    mt = jnp.broadcast_to(mask_token[None, :].astype(jnp.float32), (MT_ROWS, D))
    table = jnp.concatenate([val2d, mt], axis=0)
    ridx = revert_idx.reshape(ROWS).astype(jnp.int32)
    mask = remain_padding_mask.reshape(B * LK).astype(jnp.int32)
    out = _revert_sc(table, ridx, mask, pos_enc.astype(jnp.float32))
    return out.reshape(B, LF, D)
